# baseline (speedup 1.0000x reference)
import functools

import jax
import jax.numpy as jnp
from jax import lax
from jax.experimental import pallas as pl
from jax.experimental.pallas import tpu as pltpu
from jax.experimental.pallas import tpu_sc as plsc

N = 10000
E = 320000
D_IN = 128
H = 64
C_OUT = 40
C_PAD = 48

NC = 2
NS = 16
NW = NC * NS
CHUNK = 128
NCHUNK = 80
E_PAD = NW * NCHUNK * CHUNK
TOT_CHUNK = E_PAD // CHUNK
NBUF = 4
K0 = 36
K1 = 160 - K0
KMAX = max(K0, K1)
N_PAD = 10240
RPT = N_PAD // NS
RPT2 = RPT // 2
DEG_W = 16
RB = 1000


def _mesh():
    return plsc.VectorSubcoreMesh(
        core_axis_name="c", subcore_axis_name="s", num_cores=NC, num_subcores=NS
    )


def _make_deg_kernel():
    @functools.partial(
        pl.kernel,
        out_type=jax.ShapeDtypeStruct((NC, N_PAD, DEG_W), jnp.float32),
        mesh=_mesh(),
        scratch_types=[
            pltpu.VMEM((NCHUNK, CHUNK), jnp.int32),
            pltpu.VMEM((CHUNK, DEG_W), jnp.float32),
            pltpu.VMEM((RPT, DEG_W), jnp.float32),
            pltpu.VMEM_SHARED((N_PAD, DEG_W), jnp.float32),
            pltpu.SemaphoreType.DMA,
        ],
        compiler_params=pltpu.CompilerParams(use_tc_tiling_on_sc=False),
    )
    def deg_kernel(dst_hbm, out_hbm, dst_v, ones_v, buf_v, acc_sh, sem):
        c = lax.axis_index("c")
        s = lax.axis_index("s")
        wid = c * NS + s
        pltpu.sync_copy(dst_hbm.at[wid], dst_v)

        def fill_ones(i, carry):
            ones_v[i, :] = jnp.ones((16,), jnp.float32)
            return carry

        lax.fori_loop(0, CHUNK, fill_ones, 0)

        def fill_zero(i, carry):
            buf_v[i, :] = jnp.zeros((16,), jnp.float32)
            return carry

        lax.fori_loop(0, RPT, fill_zero, 0)
        pltpu.sync_copy(buf_v, acc_sh.at[pl.ds(s * RPT, RPT)])
        plsc.subcore_barrier()

        def step(j, carry):
            pltpu.sync_copy(ones_v, acc_sh.at[dst_v.at[j]], add=True)
            return carry

        lax.fori_loop(0, NCHUNK, step, 0)
        plsc.subcore_barrier()
        pltpu.sync_copy(acc_sh.at[pl.ds(s * RPT, RPT)], buf_v)
        pltpu.sync_copy(buf_v, out_hbm.at[c, pl.ds(s * RPT, RPT)])

    return deg_kernel


def _make_scatter_kernel(D):
    @functools.partial(
        pl.kernel,
        out_type=jax.ShapeDtypeStruct((NC, N_PAD, D), jnp.float32),
        mesh=_mesh(),
        scratch_types=[
            pltpu.VMEM((KMAX, CHUNK), jnp.int32),
            pltpu.VMEM((KMAX, CHUNK), jnp.int32),
            pltpu.VMEM((NBUF, CHUNK, D), jnp.float32),
            pltpu.VMEM((RPT2, D), jnp.float32),
            pltpu.VMEM_SHARED((N_PAD, D), jnp.float32),
        ] + [pltpu.SemaphoreType.DMA] * NBUF,
        compiler_params=pltpu.CompilerParams(use_tc_tiling_on_sc=False),
    )
    def scat_kernel(src_hbm, dst_hbm, g_hbm, out_hbm,
                    src_v, dst_v, rows_v, buf_v, acc_sh, *sems):
        c = lax.axis_index("c")
        s = lax.axis_index("s")

        def fill_zero(i, carry):
            for j in range(D // 16):
                buf_v[i, pl.ds(j * 16, 16)] = jnp.zeros((16,), jnp.float32)
            return carry

        lax.fori_loop(0, RPT2, fill_zero, 0)
        for h in range(2):
            pltpu.sync_copy(buf_v, acc_sh.at[pl.ds(s * RPT + h * RPT2, RPT2)])
        plsc.subcore_barrier()

        def run_edges(start, K):
            pltpu.sync_copy(src_hbm.at[pl.ds(start, K)],
                            src_v.at[pl.ds(0, K)])
            pltpu.sync_copy(dst_hbm.at[pl.ds(start, K)],
                            dst_v.at[pl.ds(0, K)])
            for b in range(NBUF):
                pltpu.async_copy(g_hbm.at[src_v.at[b]], rows_v.at[b], sems[b])

            def step(j0, carry):
                for b in range(NBUF):
                    j = j0 * NBUF + b
                    pltpu.make_async_copy(
                        g_hbm.at[src_v.at[j]], rows_v.at[b], sems[b]).wait()
                    pltpu.sync_copy(rows_v.at[b], acc_sh.at[dst_v.at[j]],
                                    add=True)

                    @pl.when(j0 < K // NBUF - 1)
                    def _():
                        pltpu.async_copy(
                            g_hbm.at[src_v.at[j + NBUF]], rows_v.at[b],
                            sems[b])

                return carry

            lax.fori_loop(0, K // NBUF, step, 0)

        @pl.when(c == 0)
        def _():
            run_edges(s * K0, K0)

        @pl.when(c == 1)
        def _():
            run_edges(NS * K0 + s * K1, K1)

        plsc.subcore_barrier()
        for h in range(2):
            pltpu.sync_copy(acc_sh.at[pl.ds(s * RPT + h * RPT2, RPT2)], buf_v)
            pltpu.sync_copy(buf_v, out_hbm.at[c, pl.ds(s * RPT + h * RPT2, RPT2)])

    return scat_kernel


def _mm1(x, W1):
    def body(x_ref, w_ref, o_ref):
        o_ref[...] = jnp.dot(x_ref[...], w_ref[...],
                             preferred_element_type=jnp.float32)

    return pl.pallas_call(
        body,
        grid=(N // RB,),
        in_specs=[
            pl.BlockSpec((RB, D_IN), lambda i: (i, 0)),
            pl.BlockSpec((D_IN, H), lambda i: (0, 0)),
        ],
        out_specs=pl.BlockSpec((RB, H), lambda i: (i, 0)),
        out_shape=jax.ShapeDtypeStruct((N, H), jnp.float32),
    )(x, W1)


def _scale1(degp, h1):
    def body(degp_ref, h1_ref, g1_ref, dinv_ref):
        deg = 1.0 + degp_ref[0, :, 0:1] + degp_ref[1, :, 0:1]
        dinv = lax.rsqrt(deg)
        dinv_ref[...] = dinv
        g1_ref[...] = h1_ref[...] * dinv

    return pl.pallas_call(
        body,
        grid=(N // RB,),
        in_specs=[
            pl.BlockSpec((NC, RB, DEG_W), lambda i: (0, i, 0)),
            pl.BlockSpec((RB, H), lambda i: (i, 0)),
        ],
        out_specs=[
            pl.BlockSpec((RB, H), lambda i: (i, 0)),
            pl.BlockSpec((RB, 1), lambda i: (i, 0)),
        ],
        out_shape=[
            jax.ShapeDtypeStruct((N, H), jnp.float32),
            jax.ShapeDtypeStruct((N, 1), jnp.float32),
        ],
    )(degp, h1)


def _combine1_mm2(s1p, g1, dinv, b1r, W2p):
    def body(sp_ref, g1_ref, dinv_ref, b1_ref, w2_ref, g2_ref):
        stot = sp_ref[0] + sp_ref[1] + g1_ref[...]
        dinv = dinv_ref[...]
        o1 = jnp.maximum(stot * dinv + b1_ref[...], 0.0)
        h2 = jnp.dot(o1, w2_ref[...], preferred_element_type=jnp.float32)
        g2_ref[...] = h2 * dinv

    return pl.pallas_call(
        body,
        grid=(N // RB,),
        in_specs=[
            pl.BlockSpec((NC, RB, H), lambda i: (0, i, 0)),
            pl.BlockSpec((RB, H), lambda i: (i, 0)),
            pl.BlockSpec((RB, 1), lambda i: (i, 0)),
            pl.BlockSpec((1, H), lambda i: (0, 0)),
            pl.BlockSpec((H, C_PAD), lambda i: (0, 0)),
        ],
        out_specs=pl.BlockSpec((RB, C_PAD), lambda i: (i, 0)),
        out_shape=jax.ShapeDtypeStruct((N, C_PAD), jnp.float32),
    )(s1p, g1, dinv, b1r, W2p)


def _final(s2p, g2, dinv, b2r):
    def body(sp_ref, g2_ref, dinv_ref, b2_ref, o_ref):
        o = (sp_ref[0] + sp_ref[1] + g2_ref[...]) * dinv_ref[...]
        o = o[:, :C_OUT] + b2_ref[...]
        m = jnp.max(o, axis=1, keepdims=True)
        lse = jnp.log(jnp.sum(jnp.exp(o - m), axis=1, keepdims=True)) + m
        o_ref[...] = o - lse

    return pl.pallas_call(
        body,
        grid=(N // RB,),
        in_specs=[
            pl.BlockSpec((NC, RB, C_PAD), lambda i: (0, i, 0)),
            pl.BlockSpec((RB, C_PAD), lambda i: (i, 0)),
            pl.BlockSpec((RB, 1), lambda i: (i, 0)),
            pl.BlockSpec((1, C_OUT), lambda i: (0, 0)),
        ],
        out_specs=pl.BlockSpec((RB, C_OUT), lambda i: (i, 0)),
        out_shape=jax.ShapeDtypeStruct((N, C_OUT), jnp.float32),
    )(s2p, g2, dinv, b2r)


def kernel(x, edge_index, W1, b1, W2, b2):
    pad = E_PAD - E
    src = jnp.concatenate(
        [edge_index[0], jnp.zeros((pad,), jnp.int32)]).reshape(TOT_CHUNK, CHUNK)
    junk = N + jnp.arange(pad, dtype=jnp.int32) % (N_PAD - N)
    dst = jnp.concatenate(
        [edge_index[1], junk]).reshape(TOT_CHUNK, CHUNK)

    degp = _make_deg_kernel()(dst.reshape(NW, NCHUNK, CHUNK))
    h1 = _mm1(x, W1)
    g1, dinv = _scale1(degp, h1)
    s1p = _make_scatter_kernel(H)(src, dst, g1)
    W2p = jnp.pad(W2, ((0, 0), (0, C_PAD - C_OUT)))
    g2 = _combine1_mm2(s1p, g1, dinv, b1.reshape(1, H), W2p)
    s2p = _make_scatter_kernel(C_PAD)(src, dst, g2)
    out = _final(s2p, g2, dinv, b2.reshape(1, C_OUT))
    return out


# speedup vs baseline: 1.0576x; 1.0576x over previous
import functools

import jax
import jax.numpy as jnp
from jax import lax
from jax.experimental import pallas as pl
from jax.experimental.pallas import tpu as pltpu
from jax.experimental.pallas import tpu_sc as plsc

N = 10000
E = 320000
D_IN = 128
H = 64
C_OUT = 40
C_PAD = 48

NC = 2
NS = 16
NW = NC * NS
CHUNK = 128
NCHUNK = 80
E_PAD = NW * NCHUNK * CHUNK
TOT_CHUNK = E_PAD // CHUNK
NBUF = 4
K0 = 124
K1 = 160 - K0
KMAX = max(K0, K1)
N_PAD = 10240
RPT = N_PAD // NS
RPT2 = RPT // 2
DEG_W = 16
RB = 1000


def _mesh():
    return plsc.VectorSubcoreMesh(
        core_axis_name="c", subcore_axis_name="s", num_cores=NC, num_subcores=NS
    )


def _make_deg_kernel():
    @functools.partial(
        pl.kernel,
        out_type=jax.ShapeDtypeStruct((NC, N_PAD, DEG_W), jnp.float32),
        mesh=_mesh(),
        scratch_types=[
            pltpu.VMEM((NCHUNK, CHUNK), jnp.int32),
            pltpu.VMEM((CHUNK, DEG_W), jnp.float32),
            pltpu.VMEM((RPT, DEG_W), jnp.float32),
            pltpu.VMEM_SHARED((N_PAD, DEG_W), jnp.float32),
            pltpu.SemaphoreType.DMA,
        ],
        compiler_params=pltpu.CompilerParams(use_tc_tiling_on_sc=False),
    )
    def deg_kernel(dst_hbm, out_hbm, dst_v, ones_v, buf_v, acc_sh, sem):
        c = lax.axis_index("c")
        s = lax.axis_index("s")
        wid = c * NS + s
        pltpu.sync_copy(dst_hbm.at[wid], dst_v)

        def fill_ones(i, carry):
            ones_v[i, :] = jnp.ones((16,), jnp.float32)
            return carry

        lax.fori_loop(0, CHUNK, fill_ones, 0)

        def fill_zero(i, carry):
            buf_v[i, :] = jnp.zeros((16,), jnp.float32)
            return carry

        lax.fori_loop(0, RPT, fill_zero, 0)
        pltpu.sync_copy(buf_v, acc_sh.at[pl.ds(s * RPT, RPT)])
        plsc.subcore_barrier()

        def step(j, carry):
            pltpu.sync_copy(ones_v, acc_sh.at[dst_v.at[j]], add=True)
            return carry

        lax.fori_loop(0, NCHUNK, step, 0)
        plsc.subcore_barrier()
        pltpu.sync_copy(acc_sh.at[pl.ds(s * RPT, RPT)], buf_v)
        pltpu.sync_copy(buf_v, out_hbm.at[c, pl.ds(s * RPT, RPT)])

    return deg_kernel


def _make_scatter_kernel(D):
    @functools.partial(
        pl.kernel,
        out_type=jax.ShapeDtypeStruct((NC, N_PAD, D), jnp.float32),
        mesh=_mesh(),
        scratch_types=[
            pltpu.VMEM((KMAX, CHUNK), jnp.int32),
            pltpu.VMEM((KMAX, CHUNK), jnp.int32),
            pltpu.VMEM((NBUF, CHUNK, D), jnp.float32),
            pltpu.VMEM((RPT2, D), jnp.float32),
            pltpu.VMEM_SHARED((N_PAD, D), jnp.float32),
        ] + [pltpu.SemaphoreType.DMA] * NBUF,
        compiler_params=pltpu.CompilerParams(use_tc_tiling_on_sc=False),
    )
    def scat_kernel(src_hbm, dst_hbm, g_hbm, out_hbm,
                    src_v, dst_v, rows_v, buf_v, acc_sh, *sems):
        c = lax.axis_index("c")
        s = lax.axis_index("s")

        def fill_zero(i, carry):
            for j in range(D // 16):
                buf_v[i, pl.ds(j * 16, 16)] = jnp.zeros((16,), jnp.float32)
            return carry

        lax.fori_loop(0, RPT2, fill_zero, 0)
        for h in range(2):
            pltpu.sync_copy(buf_v, acc_sh.at[pl.ds(s * RPT + h * RPT2, RPT2)])
        plsc.subcore_barrier()

        def run_edges(start, K):
            pltpu.sync_copy(src_hbm.at[pl.ds(start, K)],
                            src_v.at[pl.ds(0, K)])
            pltpu.sync_copy(dst_hbm.at[pl.ds(start, K)],
                            dst_v.at[pl.ds(0, K)])
            for b in range(NBUF):
                pltpu.async_copy(g_hbm.at[src_v.at[b]], rows_v.at[b], sems[b])

            def step(j0, carry):
                for b in range(NBUF):
                    j = j0 * NBUF + b
                    pltpu.make_async_copy(
                        g_hbm.at[src_v.at[j]], rows_v.at[b], sems[b]).wait()
                    pltpu.sync_copy(rows_v.at[b], acc_sh.at[dst_v.at[j]],
                                    add=True)

                    @pl.when(j0 < K // NBUF - 1)
                    def _():
                        pltpu.async_copy(
                            g_hbm.at[src_v.at[j + NBUF]], rows_v.at[b],
                            sems[b])

                return carry

            lax.fori_loop(0, K // NBUF, step, 0)

        @pl.when(c == 0)
        def _():
            run_edges(s * K0, K0)

        @pl.when(c == 1)
        def _():
            run_edges(NS * K0 + s * K1, K1)

        plsc.subcore_barrier()
        for h in range(2):
            pltpu.sync_copy(acc_sh.at[pl.ds(s * RPT + h * RPT2, RPT2)], buf_v)
            pltpu.sync_copy(buf_v, out_hbm.at[c, pl.ds(s * RPT + h * RPT2, RPT2)])

    return scat_kernel


def _mm1(x, W1):
    def body(x_ref, w_ref, o_ref):
        o_ref[...] = jnp.dot(x_ref[...], w_ref[...],
                             preferred_element_type=jnp.float32)

    return pl.pallas_call(
        body,
        grid=(N // RB,),
        in_specs=[
            pl.BlockSpec((RB, D_IN), lambda i: (i, 0)),
            pl.BlockSpec((D_IN, H), lambda i: (0, 0)),
        ],
        out_specs=pl.BlockSpec((RB, H), lambda i: (i, 0)),
        out_shape=jax.ShapeDtypeStruct((N, H), jnp.float32),
    )(x, W1)


def _scale1(degp, h1):
    def body(degp_ref, h1_ref, g1_ref, dinv_ref):
        deg = 1.0 + degp_ref[0, :, 0:1] + degp_ref[1, :, 0:1]
        dinv = lax.rsqrt(deg)
        dinv_ref[...] = dinv
        g1_ref[...] = h1_ref[...] * dinv

    return pl.pallas_call(
        body,
        grid=(N // RB,),
        in_specs=[
            pl.BlockSpec((NC, RB, DEG_W), lambda i: (0, i, 0)),
            pl.BlockSpec((RB, H), lambda i: (i, 0)),
        ],
        out_specs=[
            pl.BlockSpec((RB, H), lambda i: (i, 0)),
            pl.BlockSpec((RB, 1), lambda i: (i, 0)),
        ],
        out_shape=[
            jax.ShapeDtypeStruct((N, H), jnp.float32),
            jax.ShapeDtypeStruct((N, 1), jnp.float32),
        ],
    )(degp, h1)


def _combine1_mm2(s1p, g1, dinv, b1r, W2p):
    def body(sp_ref, g1_ref, dinv_ref, b1_ref, w2_ref, g2_ref):
        stot = sp_ref[0] + sp_ref[1] + g1_ref[...]
        dinv = dinv_ref[...]
        o1 = jnp.maximum(stot * dinv + b1_ref[...], 0.0)
        h2 = jnp.dot(o1, w2_ref[...], preferred_element_type=jnp.float32)
        g2_ref[...] = h2 * dinv

    return pl.pallas_call(
        body,
        grid=(N // RB,),
        in_specs=[
            pl.BlockSpec((NC, RB, H), lambda i: (0, i, 0)),
            pl.BlockSpec((RB, H), lambda i: (i, 0)),
            pl.BlockSpec((RB, 1), lambda i: (i, 0)),
            pl.BlockSpec((1, H), lambda i: (0, 0)),
            pl.BlockSpec((H, C_PAD), lambda i: (0, 0)),
        ],
        out_specs=pl.BlockSpec((RB, C_PAD), lambda i: (i, 0)),
        out_shape=jax.ShapeDtypeStruct((N, C_PAD), jnp.float32),
    )(s1p, g1, dinv, b1r, W2p)


def _final(s2p, g2, dinv, b2r):
    def body(sp_ref, g2_ref, dinv_ref, b2_ref, o_ref):
        o = (sp_ref[0] + sp_ref[1] + g2_ref[...]) * dinv_ref[...]
        o = o[:, :C_OUT] + b2_ref[...]
        m = jnp.max(o, axis=1, keepdims=True)
        lse = jnp.log(jnp.sum(jnp.exp(o - m), axis=1, keepdims=True)) + m
        o_ref[...] = o - lse

    return pl.pallas_call(
        body,
        grid=(N // RB,),
        in_specs=[
            pl.BlockSpec((NC, RB, C_PAD), lambda i: (0, i, 0)),
            pl.BlockSpec((RB, C_PAD), lambda i: (i, 0)),
            pl.BlockSpec((RB, 1), lambda i: (i, 0)),
            pl.BlockSpec((1, C_OUT), lambda i: (0, 0)),
        ],
        out_specs=pl.BlockSpec((RB, C_OUT), lambda i: (i, 0)),
        out_shape=jax.ShapeDtypeStruct((N, C_OUT), jnp.float32),
    )(s2p, g2, dinv, b2r)


def kernel(x, edge_index, W1, b1, W2, b2):
    pad = E_PAD - E
    src = jnp.concatenate(
        [edge_index[0], jnp.zeros((pad,), jnp.int32)]).reshape(TOT_CHUNK, CHUNK)
    junk = N + jnp.arange(pad, dtype=jnp.int32) % (N_PAD - N)
    dst = jnp.concatenate(
        [edge_index[1], junk]).reshape(TOT_CHUNK, CHUNK)

    degp = _make_deg_kernel()(dst.reshape(NW, NCHUNK, CHUNK))
    h1 = _mm1(x, W1)
    g1, dinv = _scale1(degp, h1)
    s1p = _make_scatter_kernel(H)(src, dst, g1)
    W2p = jnp.pad(W2, ((0, 0), (0, C_PAD - C_OUT)))
    g2 = _combine1_mm2(s1p, g1, dinv, b1.reshape(1, H), W2p)
    s2p = _make_scatter_kernel(C_PAD)(src, dst, g2)
    out = _final(s2p, g2, dinv, b2.reshape(1, C_OUT))
    return out


# speedup vs baseline: 1.8418x; 1.7415x over previous
import functools

import jax
import jax.numpy as jnp
from jax import lax
from jax.experimental import pallas as pl
from jax.experimental.pallas import tpu as pltpu
from jax.experimental.pallas import tpu_sc as plsc

N = 10000
E = 320000
D_IN = 128
H = 64
C_OUT = 40

NC = 2
NS = 16
NW = NC * NS
CHUNK = 128
NCHUNK = 80
E_PAD = NW * NCHUNK * CHUNK
TOT_CHUNK = E_PAD // CHUNK
KTILE = TOT_CHUNK // NS
NBUF = 4
N_PAD = 10240
RPT = N_PAD // NS
RPT2 = RPT // 2
DEG_W = 16
DH = H // 2
RB = 1024
RBF = 1000


def _mesh():
    return plsc.VectorSubcoreMesh(
        core_axis_name="c", subcore_axis_name="s", num_cores=NC, num_subcores=NS
    )


def _make_deg_kernel():
    @functools.partial(
        pl.kernel,
        out_type=jax.ShapeDtypeStruct((NC, N_PAD, DEG_W), jnp.float32),
        mesh=_mesh(),
        scratch_types=[
            pltpu.VMEM((NCHUNK, CHUNK), jnp.int32),
            pltpu.VMEM((CHUNK, DEG_W), jnp.float32),
            pltpu.VMEM((RPT, DEG_W), jnp.float32),
            pltpu.VMEM_SHARED((N_PAD, DEG_W), jnp.float32),
            pltpu.SemaphoreType.DMA,
        ],
        compiler_params=pltpu.CompilerParams(use_tc_tiling_on_sc=False),
    )
    def deg_kernel(dst_hbm, out_hbm, dst_v, ones_v, buf_v, acc_sh, sem):
        c = lax.axis_index("c")
        s = lax.axis_index("s")
        wid = c * NS + s
        pltpu.sync_copy(dst_hbm.at[wid], dst_v)

        def fill_ones(i, carry):
            ones_v[i, :] = jnp.ones((16,), jnp.float32)
            return carry

        lax.fori_loop(0, CHUNK, fill_ones, 0)

        def fill_zero(i, carry):
            buf_v[i, :] = jnp.zeros((16,), jnp.float32)
            return carry

        lax.fori_loop(0, RPT, fill_zero, 0)
        pltpu.sync_copy(buf_v, acc_sh.at[pl.ds(s * RPT, RPT)])
        plsc.subcore_barrier()

        def step(j, carry):
            pltpu.sync_copy(ones_v, acc_sh.at[dst_v.at[j]], add=True)
            return carry

        lax.fori_loop(0, NCHUNK, step, 0)
        plsc.subcore_barrier()
        pltpu.sync_copy(acc_sh.at[pl.ds(s * RPT, RPT)], buf_v)
        pltpu.sync_copy(buf_v, out_hbm.at[c, pl.ds(s * RPT, RPT)])

    return deg_kernel


def _make_scatter_kernel():
    @functools.partial(
        pl.kernel,
        out_type=jax.ShapeDtypeStruct((NC, N_PAD, DH), jnp.float32),
        mesh=_mesh(),
        scratch_types=[
            pltpu.VMEM((KTILE, CHUNK), jnp.int32),
            pltpu.VMEM((KTILE, CHUNK), jnp.int32),
            pltpu.VMEM((NBUF, CHUNK, DH), jnp.float32),
            pltpu.VMEM((RPT2, DH), jnp.float32),
            pltpu.VMEM_SHARED((N_PAD, DH), jnp.float32),
            pltpu.VMEM_SHARED((N_PAD, DH), jnp.float32),
        ] + [pltpu.SemaphoreType.DMA] * NBUF,
        compiler_params=pltpu.CompilerParams(use_tc_tiling_on_sc=False),
    )
    def scat_kernel(src_hbm, dst_hbm, g_hbm, out_hbm,
                    src_v, dst_v, rows_v, buf_v, gsh, acc_sh, *sems):
        c = lax.axis_index("c")
        s = lax.axis_index("s")

        pltpu.sync_copy(g_hbm.at[c, pl.ds(s * RPT, RPT)],
                        gsh.at[pl.ds(s * RPT, RPT)])

        def fill_zero(i, carry):
            for j in range(DH // 16):
                buf_v[i, pl.ds(j * 16, 16)] = jnp.zeros((16,), jnp.float32)
            return carry

        lax.fori_loop(0, RPT2, fill_zero, 0)
        for h in range(2):
            pltpu.sync_copy(buf_v, acc_sh.at[pl.ds(s * RPT + h * RPT2, RPT2)])

        start = s * KTILE
        pltpu.sync_copy(src_hbm.at[pl.ds(start, KTILE)], src_v)
        pltpu.sync_copy(dst_hbm.at[pl.ds(start, KTILE)], dst_v)
        plsc.subcore_barrier()

        for b in range(NBUF):
            pltpu.async_copy(gsh.at[src_v.at[b]], rows_v.at[b], sems[b])

        def step(j0, carry):
            for b in range(NBUF):
                j = j0 * NBUF + b
                pltpu.make_async_copy(
                    gsh.at[src_v.at[j]], rows_v.at[b], sems[b]).wait()
                pltpu.sync_copy(rows_v.at[b], acc_sh.at[dst_v.at[j]],
                                add=True)

                @pl.when(j0 < KTILE // NBUF - 1)
                def _():
                    pltpu.async_copy(
                        gsh.at[src_v.at[j + NBUF]], rows_v.at[b], sems[b])

            return carry

        lax.fori_loop(0, KTILE // NBUF, step, 0)
        plsc.subcore_barrier()
        for h in range(2):
            pltpu.sync_copy(acc_sh.at[pl.ds(s * RPT + h * RPT2, RPT2)], buf_v)
            pltpu.sync_copy(buf_v, out_hbm.at[c, pl.ds(s * RPT + h * RPT2,
                                                       RPT2)])

    return scat_kernel


def _mm1(x, W1):
    def body(x_ref, w_ref, o_ref):
        o_ref[...] = jnp.dot(x_ref[...], w_ref[...],
                             preferred_element_type=jnp.float32)

    return pl.pallas_call(
        body,
        grid=(N_PAD // RB,),
        in_specs=[
            pl.BlockSpec((RB, D_IN), lambda i: (i, 0)),
            pl.BlockSpec((D_IN, H), lambda i: (0, 0)),
        ],
        out_specs=pl.BlockSpec((RB, H), lambda i: (i, 0)),
        out_shape=jax.ShapeDtypeStruct((N_PAD, H), jnp.float32),
    )(x, W1)


def _scale1(degp, h1):
    def body(degp_ref, h1_ref, g1_ref, dinv_ref):
        deg = 1.0 + degp_ref[0, :, 0:1] + degp_ref[1, :, 0:1]
        dinv = lax.rsqrt(deg)
        dinv_ref[...] = dinv
        g = h1_ref[...] * dinv
        g1_ref[0] = g[:, :DH]
        g1_ref[1] = g[:, DH:]

    return pl.pallas_call(
        body,
        grid=(N_PAD // RB,),
        in_specs=[
            pl.BlockSpec((NC, RB, DEG_W), lambda i: (0, i, 0)),
            pl.BlockSpec((RB, H), lambda i: (i, 0)),
        ],
        out_specs=[
            pl.BlockSpec((NC, RB, DH), lambda i: (0, i, 0)),
            pl.BlockSpec((RB, 1), lambda i: (i, 0)),
        ],
        out_shape=[
            jax.ShapeDtypeStruct((NC, N_PAD, DH), jnp.float32),
            jax.ShapeDtypeStruct((N_PAD, 1), jnp.float32),
        ],
    )(degp, h1)


def _combine1_mm2(s1p, g1s, dinv, b1r, W2p):
    def body(sp_ref, g1_ref, dinv_ref, b1_ref, w2_ref, g2_ref):
        stot = jnp.concatenate(
            [sp_ref[0] + g1_ref[0], sp_ref[1] + g1_ref[1]], axis=1)
        dinv = dinv_ref[...]
        o1 = jnp.maximum(stot * dinv + b1_ref[...], 0.0)
        h2 = jnp.dot(o1, w2_ref[...], preferred_element_type=jnp.float32)
        g2 = h2 * dinv
        g2_ref[0] = g2[:, :DH]
        g2_ref[1] = g2[:, DH:]

    return pl.pallas_call(
        body,
        grid=(N_PAD // RB,),
        in_specs=[
            pl.BlockSpec((NC, RB, DH), lambda i: (0, i, 0)),
            pl.BlockSpec((NC, RB, DH), lambda i: (0, i, 0)),
            pl.BlockSpec((RB, 1), lambda i: (i, 0)),
            pl.BlockSpec((1, H), lambda i: (0, 0)),
            pl.BlockSpec((H, H), lambda i: (0, 0)),
        ],
        out_specs=pl.BlockSpec((NC, RB, DH), lambda i: (0, i, 0)),
        out_shape=jax.ShapeDtypeStruct((NC, N_PAD, DH), jnp.float32),
    )(s1p, g1s, dinv, b1r, W2p)


def _final(s2p, g2s, dinv, b2r):
    def body(sp_ref, g2_ref, dinv_ref, b2_ref, o_ref):
        o = jnp.concatenate(
            [sp_ref[0] + g2_ref[0], sp_ref[1] + g2_ref[1]], axis=1)
        o = o * dinv_ref[...]
        o = o[:, :C_OUT] + b2_ref[...]
        m = jnp.max(o, axis=1, keepdims=True)
        lse = jnp.log(jnp.sum(jnp.exp(o - m), axis=1, keepdims=True)) + m
        o_ref[...] = o - lse

    return pl.pallas_call(
        body,
        grid=(N // RBF,),
        in_specs=[
            pl.BlockSpec((NC, RBF, DH), lambda i: (0, i, 0)),
            pl.BlockSpec((NC, RBF, DH), lambda i: (0, i, 0)),
            pl.BlockSpec((RBF, 1), lambda i: (i, 0)),
            pl.BlockSpec((1, C_OUT), lambda i: (0, 0)),
        ],
        out_specs=pl.BlockSpec((RBF, C_OUT), lambda i: (i, 0)),
        out_shape=jax.ShapeDtypeStruct((N, C_OUT), jnp.float32),
    )(s2p, g2s, dinv, b2r)


def kernel(x, edge_index, W1, b1, W2, b2):
    pad = E_PAD - E
    src = jnp.concatenate(
        [edge_index[0], jnp.zeros((pad,), jnp.int32)]).reshape(TOT_CHUNK, CHUNK)
    junk = N + jnp.arange(pad, dtype=jnp.int32) % (N_PAD - N)
    dst = jnp.concatenate(
        [edge_index[1], junk]).reshape(TOT_CHUNK, CHUNK)
    x_p = jnp.pad(x, ((0, N_PAD - N), (0, 0)))

    degp = _make_deg_kernel()(dst.reshape(NW, NCHUNK, CHUNK))
    h1 = _mm1(x_p, W1)
    g1s, dinv = _scale1(degp, h1)
    scat = _make_scatter_kernel()
    s1p = scat(src, dst, g1s)
    W2p = jnp.pad(W2, ((0, 0), (0, H - C_OUT)))
    g2s = _combine1_mm2(s1p, g1s, dinv, b1.reshape(1, H), W2p)
    s2p = scat(src, dst, g2s)
    out = _final(s2p, g2s, dinv, b2.reshape(1, C_OUT))
    return out
